# Initial kernel scaffold; baseline (speedup 1.0000x reference)
import jax
import jax.numpy as jnp
from jax.experimental import pallas as pl


def kernel(x, edge_index, edge_attr, W1, a_src1, a_dst1, We1, a_e1, b1, W2, a_src2, a_dst2, We2, a_e2, b2):
    raise NotImplementedError("write your pallas kernel here")


# speedup vs baseline: 13.2056x; 13.2056x over previous
import functools

import jax
import jax.numpy as jnp
from jax import lax
from jax.experimental import pallas as pl
from jax.experimental.pallas import tpu as pltpu
from jax.experimental.pallas import tpu_sc as plsc

F32 = jnp.float32
NC = 2
NS = 16
NT = NC * NS
LANES = 16


def _round_up(a, m):
    return (a + m - 1) // m * m


def _node_pass(xp, W, a_src, a_dst, blk=1024):
    Np, D = xp.shape
    grid = Np // blk

    def body(x_ref, w_ref, as_ref, ad_ref, h_ref, sd_ref, ms_ref, md_ref):
        i = pl.program_id(0)
        h = jnp.dot(x_ref[...], w_ref[...], preferred_element_type=F32)
        h_ref[...] = h
        s = jnp.sum(h * as_ref[...][None, :], axis=1)
        d = jnp.sum(h * ad_ref[...][None, :], axis=1)
        sd_ref[...] = jnp.concatenate([s[None, :], d[None, :]], axis=0)
        ninf = jnp.full((1, 1), -jnp.inf, F32)
        ms = jnp.full((1, 1), jnp.max(s), F32)
        md = jnp.full((1, 1), jnp.max(d), F32)
        ms_ref[...] = jnp.maximum(jnp.where(i == 0, ninf, ms_ref[...]), ms)
        md_ref[...] = jnp.maximum(jnp.where(i == 0, ninf, md_ref[...]), md)

    return pl.pallas_call(
        body,
        grid=(grid,),
        in_specs=[
            pl.BlockSpec((blk, D), lambda i: (i, 0)),
            pl.BlockSpec((D, D), lambda i: (0, 0)),
            pl.BlockSpec((D,), lambda i: (0,)),
            pl.BlockSpec((D,), lambda i: (0,)),
        ],
        out_specs=[
            pl.BlockSpec((blk, D), lambda i: (i, 0)),
            pl.BlockSpec((2, blk), lambda i: (0, i)),
            pl.BlockSpec((1, 1), lambda i: (0, 0)),
            pl.BlockSpec((1, 1), lambda i: (0, 0)),
        ],
        out_shape=[
            jax.ShapeDtypeStruct((Np, D), F32),
            jax.ShapeDtypeStruct((2, Np), F32),
            jax.ShapeDtypeStruct((1, 1), F32),
            jax.ShapeDtypeStruct((1, 1), F32),
        ],
    )(xp, W, a_src, a_dst)


def _edge_pass(edge_attr, We, a_e, blk=6400):
    E, De = edge_attr.shape
    D = We.shape[1]
    grid = E // blk

    def body(ea_ref, we_ref, ae_ref, e_ref, mx_ref):
        i = pl.program_id(0)
        he = jnp.dot(ea_ref[...], we_ref[...], preferred_element_type=F32)
        ev = jnp.sum(he * ae_ref[...][None, :], axis=1)
        e_ref[...] = ev[None, :]
        mx = jnp.full((1, 1), jnp.max(ev), F32)
        prev = jnp.where(i == 0, jnp.full((1, 1), -jnp.inf, F32), mx_ref[...])
        mx_ref[...] = jnp.maximum(prev, mx)

    return pl.pallas_call(
        body,
        grid=(grid,),
        in_specs=[
            pl.BlockSpec((blk, De), lambda i: (i, 0)),
            pl.BlockSpec((De, D), lambda i: (0, 0)),
            pl.BlockSpec((D,), lambda i: (0,)),
        ],
        out_specs=[
            pl.BlockSpec((1, blk), lambda i: (0, i)),
            pl.BlockSpec((1, 1), lambda i: (0, 0)),
        ],
        out_shape=[
            jax.ShapeDtypeStruct((1, E), F32),
            jax.ShapeDtypeStruct((1, 1), F32),
        ],
    )(edge_attr, We, a_e)


def _sc_aggregate(h, s, d, ef3, src3, dst3, avec, Np, D, EPT, KB):
    mesh = plsc.VectorSubcoreMesh(
        core_axis_name="c", subcore_axis_name="s", num_cores=NC, num_subcores=NS
    )
    rows_per_tile = Np // NS
    n_zero_chunks = rows_per_tile // 128
    NSB = KB // 8

    @functools.partial(
        pl.kernel,
        out_type=[
            jax.ShapeDtypeStruct((NC, Np, D), F32),
            jax.ShapeDtypeStruct((NC, Np), F32),
        ],
        mesh=mesh,
        compiler_params=pltpu.CompilerParams(needs_layout_passes=False),
        scratch_types=[
            pltpu.VMEM((Np,), F32),
            pltpu.VMEM((Np,), F32),
            pltpu.VMEM((8, 128), F32),
            pltpu.VMEM((8, 128), jnp.int32),
            pltpu.VMEM((8, 128), jnp.int32),
            pltpu.VMEM((128, D), F32),
            pltpu.VMEM((128,), F32),
            pltpu.VMEM((LANES,), F32),
            pltpu.VMEM_SHARED((Np, D), F32),
            pltpu.VMEM_SHARED((Np,), F32),
            pltpu.SemaphoreType.DMA,
        ],
    )
    def k(h_hbm, s_hbm, d_hbm, ef3_hbm, src3_hbm, dst3_hbm, av_hbm,
          acc_hbm, den_hbm,
          s_t, d_t, ef_t, src_t, dst_t, rows, wbuf, av_t,
          acc_sh, den_sh, sem):
        cid = lax.axis_index("c")
        sid = lax.axis_index("s")
        wid = cid * NS + sid

        pltpu.sync_copy(s_hbm, s_t)
        pltpu.sync_copy(d_hbm, d_t)
        pltpu.sync_copy(av_hbm, av_t)

        zeros16 = jnp.zeros((LANES,), F32)

        def zero_rows(i, carry):
            for f in range(D // LANES):
                rows[i, pl.ds(f * LANES, LANES)] = zeros16
            return carry

        lax.fori_loop(0, 128, zero_rows, 0)

        for t in range(n_zero_chunks):
            off = sid * rows_per_tile + t * 128
            pltpu.sync_copy(rows, acc_sh.at[pl.ds(off, 128)])
            pltpu.sync_copy(rows.at[0], den_sh.at[pl.ds(off, 128)])
        plsc.subcore_barrier()

        av = av_t[...]

        def sb_body(sb, carry):
            sbase = sb * 8
            pltpu.sync_copy(ef3_hbm.at[wid, pl.ds(sbase, 8)], ef_t)
            pltpu.sync_copy(src3_hbm.at[wid, pl.ds(sbase, 8)], src_t)
            pltpu.sync_copy(dst3_hbm.at[wid, pl.ds(sbase, 8)], dst_t)

            for bb in range(8):
                pltpu.async_copy(h_hbm.at[src_t.at[bb]], rows, sem).wait()
                for j in range(8):
                    off = j * LANES
                    sr = src_t[bb, pl.ds(off, LANES)]
                    dr = dst_t[bb, pl.ds(off, LANES)]
                    ev = ef_t[bb, pl.ds(off, LANES)]
                    sg = plsc.load_gather(s_t, [sr])
                    dg = plsc.load_gather(d_t, [dr])
                    al = sg + dg + ev
                    al = jnp.where(al >= 0.0, al, al * F32(0.2))
                    w = jnp.exp(al - av)
                    wbuf[pl.ds(off, LANES)] = w

                def scale_body(i, c3):
                    wv = plsc.load_gather(wbuf, [jnp.full((LANES,), i, jnp.int32)])
                    for f in range(D // LANES):
                        rows[i, pl.ds(f * LANES, LANES)] = (
                            rows[i, pl.ds(f * LANES, LANES)] * wv
                        )
                    return c3

                lax.fori_loop(0, 128, scale_body, 0)
                pltpu.sync_copy(rows, acc_sh.at[dst_t.at[bb]], add=True)
                pltpu.sync_copy(wbuf, den_sh.at[dst_t.at[bb]], add=True)
            return carry

        lax.fori_loop(0, NSB, sb_body, 0)
        plsc.subcore_barrier()

        for t in range(n_zero_chunks):
            off = sid * rows_per_tile + t * 128
            pltpu.sync_copy(acc_sh.at[pl.ds(off, 128)],
                            acc_hbm.at[cid, pl.ds(off, 128)])
        off2 = sid * rows_per_tile
        pltpu.sync_copy(den_sh.at[pl.ds(off2, rows_per_tile)],
                        den_hbm.at[cid, pl.ds(off2, rows_per_tile)])

    return k(h, s, d, ef3, src3, dst3, avec)


def _combine(accp, denp, b, relu, blk=1024):
    _, Np, D = accp.shape

    def body(a_ref, den_ref, b_ref, o_ref):
        a = a_ref[0] + a_ref[1]
        dsum = jnp.sum(den_ref[...], axis=0)
        o = a / (dsum[:, None] + F32(1e-16)) + b_ref[...][None, :]
        if relu:
            o = jnp.maximum(o, F32(0.0))
        o_ref[...] = o

    grid = Np // blk
    return pl.pallas_call(
        body,
        grid=(grid,),
        in_specs=[
            pl.BlockSpec((NC, blk, D), lambda i: (0, i, 0)),
            pl.BlockSpec((NC, blk), lambda i: (0, i)),
            pl.BlockSpec((D,), lambda i: (0,)),
        ],
        out_specs=pl.BlockSpec((blk, D), lambda i: (i, 0)),
        out_shape=jax.ShapeDtypeStruct((Np, D), F32),
    )(accp, denp, b)


def _layer(xp, src3, dst3, E, Ep, edge_attr, W, a_src, a_dst, We, a_e, b,
           relu):
    Np, D = xp.shape
    EPT = Ep // NT
    KB = EPT // 128
    h, sd, mx_s, mx_d = _node_pass(xp, W, a_src, a_dst)
    e2, mx_e = _edge_pass(edge_attr, We, a_e)
    A = jnp.maximum(mx_s[0, 0] + mx_d[0, 0] + mx_e[0, 0], F32(0.0))
    avec = jnp.full((LANES,), A, F32)
    ef3 = jnp.concatenate(
        [e2[0], jnp.full((Ep - E,), -1e30, F32)]).reshape(NT, KB, 128)
    accp, denp = _sc_aggregate(h, sd[0], sd[1], ef3, src3, dst3, avec,
                               Np, D, EPT, KB)
    return _combine(accp, denp, b, relu)


def kernel(x, edge_index, edge_attr, W1, a_src1, a_dst1, We1, a_e1, b1,
           W2, a_src2, a_dst2, We2, a_e2, b2):
    N, D = x.shape
    E = edge_index.shape[1]
    Np = _round_up(N, NS * 128)
    EPT = _round_up((E + NT - 1) // NT, 1024)
    Ep = EPT * NT
    KB = EPT // 128

    xp = jnp.pad(x, ((0, Np - N), (0, 0)))
    src = edge_index[0]
    dst = edge_index[1]
    src3 = jnp.pad(src, (0, Ep - E)).reshape(NT, KB, 128)
    dst3 = jnp.pad(dst, (0, Ep - E)).reshape(NT, KB, 128)

    o1 = _layer(xp, src3, dst3, E, Ep, edge_attr,
                W1, a_src1, a_dst1, We1, a_e1, b1, relu=True)
    o2 = _layer(o1, src3, dst3, E, Ep, edge_attr,
                W2, a_src2, a_dst2, We2, a_e2, b2, relu=False)
    return o2[:N]


# speedup vs baseline: 18.4861x; 1.3999x over previous
import functools

import jax
import jax.numpy as jnp
from jax import lax
from jax.experimental import pallas as pl
from jax.experimental.pallas import tpu as pltpu
from jax.experimental.pallas import tpu_sc as plsc

F32 = jnp.float32
NC = 2
NS = 16
NT = NC * NS
LANES = 16


def _round_up(a, m):
    return (a + m - 1) // m * m


def _node_pass(xp, W, a_src, a_dst, blk=1024):
    Np, D = xp.shape
    grid = Np // blk

    def body(x_ref, w_ref, as_ref, ad_ref, h0_ref, h1_ref, sd_ref, ms_ref, md_ref):
        i = pl.program_id(0)
        h = jnp.dot(x_ref[...], w_ref[...], preferred_element_type=F32)
        h0_ref[...] = h[:, : D // 2]
        h1_ref[...] = h[:, D // 2:]
        s = jnp.sum(h * as_ref[...][None, :], axis=1)
        d = jnp.sum(h * ad_ref[...][None, :], axis=1)
        sd_ref[...] = jnp.concatenate([s[None, :], d[None, :]], axis=0)
        ninf = jnp.full((1, 1), -jnp.inf, F32)
        ms = jnp.full((1, 1), jnp.max(s), F32)
        md = jnp.full((1, 1), jnp.max(d), F32)
        ms_ref[...] = jnp.maximum(jnp.where(i == 0, ninf, ms_ref[...]), ms)
        md_ref[...] = jnp.maximum(jnp.where(i == 0, ninf, md_ref[...]), md)

    return pl.pallas_call(
        body,
        grid=(grid,),
        in_specs=[
            pl.BlockSpec((blk, D), lambda i: (i, 0)),
            pl.BlockSpec((D, D), lambda i: (0, 0)),
            pl.BlockSpec((D,), lambda i: (0,)),
            pl.BlockSpec((D,), lambda i: (0,)),
        ],
        out_specs=[
            pl.BlockSpec((blk, D // 2), lambda i: (i, 0)),
            pl.BlockSpec((blk, D // 2), lambda i: (i, 0)),
            pl.BlockSpec((2, blk), lambda i: (0, i)),
            pl.BlockSpec((1, 1), lambda i: (0, 0)),
            pl.BlockSpec((1, 1), lambda i: (0, 0)),
        ],
        out_shape=[
            jax.ShapeDtypeStruct((Np, D // 2), F32),
            jax.ShapeDtypeStruct((Np, D // 2), F32),
            jax.ShapeDtypeStruct((2, Np), F32),
            jax.ShapeDtypeStruct((1, 1), F32),
            jax.ShapeDtypeStruct((1, 1), F32),
        ],
    )(xp, W, a_src, a_dst)


def _edge_pass(edge_attr, We, a_e, blk=6400):
    E, De = edge_attr.shape
    D = We.shape[1]
    grid = E // blk

    def body(ea_ref, we_ref, ae_ref, e_ref, mx_ref):
        i = pl.program_id(0)
        he = jnp.dot(ea_ref[...], we_ref[...], preferred_element_type=F32)
        ev = jnp.sum(he * ae_ref[...][None, :], axis=1)
        e_ref[...] = ev[None, :]
        mx = jnp.full((1, 1), jnp.max(ev), F32)
        prev = jnp.where(i == 0, jnp.full((1, 1), -jnp.inf, F32), mx_ref[...])
        mx_ref[...] = jnp.maximum(prev, mx)

    return pl.pallas_call(
        body,
        grid=(grid,),
        in_specs=[
            pl.BlockSpec((blk, De), lambda i: (i, 0)),
            pl.BlockSpec((De, D), lambda i: (0, 0)),
            pl.BlockSpec((D,), lambda i: (0,)),
        ],
        out_specs=[
            pl.BlockSpec((1, blk), lambda i: (0, i)),
            pl.BlockSpec((1, 1), lambda i: (0, 0)),
        ],
        out_shape=[
            jax.ShapeDtypeStruct((1, E), F32),
            jax.ShapeDtypeStruct((1, 1), F32),
        ],
    )(edge_attr, We, a_e)


def _sc_aggregate(h0, h1, s, d, ef3, src3, dst3, avec, Np, D, EPT, KB):
    mesh = plsc.VectorSubcoreMesh(
        core_axis_name="c", subcore_axis_name="s", num_cores=NC, num_subcores=NS
    )
    Dh = D // 2
    rows_per_tile = Np // NS
    n_zero_chunks = rows_per_tile // 128
    SBB = 32
    STAGES = KB // SBB

    @functools.partial(
        pl.kernel,
        out_type=[
            jax.ShapeDtypeStruct((NC, Np, Dh), F32),
            jax.ShapeDtypeStruct((NC, Np), F32),
        ],
        mesh=mesh,
        compiler_params=pltpu.CompilerParams(
            needs_layout_passes=False, use_tc_tiling_on_sc=False),
        scratch_types=[
            pltpu.VMEM((Np,), F32),
            pltpu.VMEM((Np,), F32),
            pltpu.VMEM((SBB, 128), F32),
            pltpu.VMEM((SBB, 128), jnp.int32),
            pltpu.VMEM((SBB, 128), jnp.int32),
            pltpu.VMEM((4, 128, Dh), F32),
            pltpu.VMEM((4, 128), F32),
            pltpu.VMEM((LANES,), F32),
            pltpu.VMEM_SHARED((Np, Dh), F32),
            pltpu.VMEM_SHARED((Np,), F32),
            pltpu.SemaphoreType.DMA,
            pltpu.SemaphoreType.DMA,
            pltpu.SemaphoreType.DMA,
        ],
    )
    def k(h0_hbm, h1_hbm, s_hbm, d_hbm, ef3_hbm, src3_hbm, dst3_hbm, av_hbm,
          acc_hbm, den_hbm,
          s_t, d_t, ef_t, src_t, dst_t, rows, wbuf, av_t,
          acc_sh, den_sh, gsem, ssem, dsem):
        cid = lax.axis_index("c")
        sid = lax.axis_index("s")

        pltpu.sync_copy(s_hbm, s_t)
        pltpu.sync_copy(d_hbm, d_t)
        pltpu.sync_copy(av_hbm, av_t)

        zeros16 = jnp.zeros((LANES,), F32)

        def zero_rows(i, carry):
            for f in range(Dh // LANES):
                rows[0, i, pl.ds(f * LANES, LANES)] = zeros16
            return carry

        lax.fori_loop(0, 128, zero_rows, 0)
        for f in range(8):
            wbuf[0, pl.ds(f * LANES, LANES)] = zeros16

        for t in range(n_zero_chunks):
            off = sid * rows_per_tile + t * 128
            pltpu.sync_copy(rows.at[0], acc_sh.at[pl.ds(off, 128)])
            pltpu.sync_copy(wbuf.at[0], den_sh.at[pl.ds(off, 128)])
        plsc.subcore_barrier()

        av = av_t[...]

        def issue_gather(lb, r):
            idx = src_t.at[lb]

            @pl.when(cid == 0)
            def _():
                pltpu.async_copy(h0_hbm.at[idx], rows.at[r], gsem)

            @pl.when(cid == 1)
            def _():
                pltpu.async_copy(h1_hbm.at[idx], rows.at[r], gsem)

        def wait_gather():
            pltpu.make_async_copy(h0_hbm.at[src_t.at[0]], rows.at[0], gsem).wait()

        def wait_row_scatter():
            pltpu.make_async_copy(rows.at[0], acc_sh.at[dst_t.at[0]], ssem).wait()

        def wait_den_scatter():
            pltpu.make_async_copy(wbuf.at[0], den_sh.at[dst_t.at[0]], dsem).wait()

        def stage_body(st, carry):
            sbase = st * SBB
            pltpu.sync_copy(ef3_hbm.at[sid, pl.ds(sbase, SBB)], ef_t)
            pltpu.sync_copy(src3_hbm.at[sid, pl.ds(sbase, SBB)], src_t)
            pltpu.sync_copy(dst3_hbm.at[sid, pl.ds(sbase, SBB)], dst_t)

            issue_gather(0, 0)
            issue_gather(1, 1)

            def batch_body(lb, c2):
                r = lax.rem(lb, 4)

                @pl.when(lb >= 2)
                def _():
                    wait_row_scatter()
                    wait_den_scatter()

                @pl.when(lb + 2 <= SBB - 1)
                def _():
                    issue_gather(lb + 2, lax.rem(lb + 2, 4))

                wait_gather()

                for j in range(8):
                    off = j * LANES
                    sr = src_t[lb, pl.ds(off, LANES)]
                    dr = dst_t[lb, pl.ds(off, LANES)]
                    ev = ef_t[lb, pl.ds(off, LANES)]
                    sg = plsc.load_gather(s_t, [sr])
                    dg = plsc.load_gather(d_t, [dr])
                    al = sg + dg + ev
                    al = jnp.where(al >= 0.0, al, al * F32(0.2))
                    w = jnp.exp(al - av)
                    wbuf[r, pl.ds(off, LANES)] = w

                def scale_body(i, c3):
                    wv = plsc.load_gather(
                        wbuf.at[r], [jnp.full((LANES,), i, jnp.int32)])
                    for f in range(Dh // LANES):
                        rows[r, i, pl.ds(f * LANES, LANES)] = (
                            rows[r, i, pl.ds(f * LANES, LANES)] * wv
                        )
                    return c3

                lax.fori_loop(0, 128, scale_body, 0)
                pltpu.async_copy(rows.at[r], acc_sh.at[dst_t.at[lb]], ssem,
                                 add=True)
                pltpu.async_copy(wbuf.at[r], den_sh.at[dst_t.at[lb]], dsem,
                                 add=True)
                return c2

            lax.fori_loop(0, SBB, batch_body, 0)
            wait_row_scatter()
            wait_den_scatter()
            wait_row_scatter()
            wait_den_scatter()
            return carry

        lax.fori_loop(0, STAGES, stage_body, 0)
        plsc.subcore_barrier()

        for t in range(n_zero_chunks):
            off = sid * rows_per_tile + t * 128
            pltpu.sync_copy(acc_sh.at[pl.ds(off, 128)],
                            acc_hbm.at[cid, pl.ds(off, 128)])
        off2 = sid * rows_per_tile
        pltpu.sync_copy(den_sh.at[pl.ds(off2, rows_per_tile)],
                        den_hbm.at[cid, pl.ds(off2, rows_per_tile)])

    return k(h0, h1, s, d, ef3, src3, dst3, avec)


def _combine(accp, denp, b, relu, blk=1024):
    _, Np, _ = accp.shape
    D = b.shape[0]

    def body(a_ref, den_ref, b_ref, o_ref):
        a = jnp.concatenate([a_ref[0], a_ref[1]], axis=1)
        dsum = den_ref[0]
        o = a / (dsum[:, None] + F32(1e-16)) + b_ref[...][None, :]
        if relu:
            o = jnp.maximum(o, F32(0.0))
        o_ref[...] = o

    grid = Np // blk
    return pl.pallas_call(
        body,
        grid=(grid,),
        in_specs=[
            pl.BlockSpec((NC, blk, D // 2), lambda i: (0, i, 0)),
            pl.BlockSpec((NC, blk), lambda i: (0, i)),
            pl.BlockSpec((D,), lambda i: (0,)),
        ],
        out_specs=pl.BlockSpec((blk, D), lambda i: (i, 0)),
        out_shape=jax.ShapeDtypeStruct((Np, D), F32),
    )(accp, denp, b)


def _layer(xp, src3, dst3, E, Ep, edge_attr, W, a_src, a_dst, We, a_e, b,
           relu):
    Np, D = xp.shape
    EPT = Ep // NS
    KB = EPT // 128
    h0, h1, sd, mx_s, mx_d = _node_pass(xp, W, a_src, a_dst)
    e2, mx_e = _edge_pass(edge_attr, We, a_e)
    A = jnp.maximum(mx_s[0, 0] + mx_d[0, 0] + mx_e[0, 0], F32(0.0))
    avec = jnp.full((LANES,), A, F32)
    ef3 = jnp.concatenate(
        [e2[0], jnp.full((Ep - E,), -1e30, F32)]).reshape(NS, KB, 128)
    accp, denp = _sc_aggregate(h0, h1, sd[0], sd[1], ef3, src3, dst3, avec,
                               Np, D, EPT, KB)
    return _combine(accp, denp, b, relu)


def kernel(x, edge_index, edge_attr, W1, a_src1, a_dst1, We1, a_e1, b1,
           W2, a_src2, a_dst2, We2, a_e2, b2):
    N, D = x.shape
    E = edge_index.shape[1]
    Np = _round_up(N, NS * 128)
    EPT = _round_up((E + NS - 1) // NS, 32 * 128)
    Ep = EPT * NS
    KB = EPT // 128

    xp = jnp.pad(x, ((0, Np - N), (0, 0)))
    src = edge_index[0]
    dst = edge_index[1]
    src3 = jnp.pad(src, (0, Ep - E)).reshape(NS, KB, 128)
    dst3 = jnp.pad(dst, (0, Ep - E)).reshape(NS, KB, 128)

    o1 = _layer(xp, src3, dst3, E, Ep, edge_attr,
                W1, a_src1, a_dst1, We1, a_e1, b1, relu=True)
    o2 = _layer(o1, src3, dst3, E, Ep, edge_attr,
                W2, a_src2, a_dst2, We2, a_e2, b2, relu=False)
    return o2[:N]
